# baseline (speedup 1.0000x reference)
import jax
import jax.numpy as jnp
from jax.experimental import pallas as pl
from jax.experimental.pallas import tpu as pltpu

_L = 4
_E = 8
_D = 80
_BLK = 512


def _top2_coeffs(g):
    T = g.shape[0]
    ids = jax.lax.broadcasted_iota(jnp.int32, (T, _E), 1)
    idf = ids.astype(jnp.float32)

    v0 = jnp.max(g, axis=1, keepdims=True)
    idx0 = jnp.min(jnp.where(g == v0, ids, _E), axis=1, keepdims=True)
    g1 = jnp.where(ids == idx0, -jnp.inf, g)
    v1 = jnp.max(g1, axis=1, keepdims=True)
    idx1 = jnp.min(jnp.where(g1 == v1, ids, _E), axis=1, keepdims=True)

    idx0f = idx0.astype(jnp.float32)
    idx1f = idx1.astype(jnp.float32)
    c = (jnp.where(v0 == idf, idx0f, 0.0)
         + jnp.where(v1 == idf, idx1f, 0.0))
    return c


def _moe_body(x_ref, wgt_ref, bg_ref, wet_ref, be_ref, o_ref, xs_ref):
    T = x_ref.shape[0]
    xs_ref[...] = x_ref[...]

    for layer in range(_L):
        x = xs_ref[...]
        g = (jnp.dot(x, wgt_ref[layer], preferred_element_type=jnp.float32)
             + bg_ref[layer][None, :])
        c = _top2_coeffs(g)
        active = jnp.any(c != 0.0)
        dst = o_ref if layer == _L - 1 else xs_ref

        @pl.when(active)
        def _dense():
            acc = jnp.zeros((T, _D), jnp.float32)
            for e in range(_E):
                eo = (jnp.dot(x, wet_ref[e], preferred_element_type=jnp.float32)
                      + be_ref[e][None, :])
                acc = acc + c[:, e:e + 1] * eo
            dst[...] = acc

        @pl.when(jnp.logical_not(active))
        def _zero():
            dst[...] = jnp.zeros((T, _D), jnp.float32)


def kernel(input_features, Wg, bg, We, be, interpret=False):
    B, S, D = input_features.shape
    N = B * S
    x = input_features.reshape(N, D)
    WgT = jnp.transpose(Wg, (0, 2, 1))
    WeT = jnp.transpose(We, (0, 2, 1))

    grid = (N // _BLK,)
    out = pl.pallas_call(
        _moe_body,
        grid=grid,
        in_specs=[
            pl.BlockSpec((_BLK, D), lambda i: (i, 0)),
            pl.BlockSpec((_L, D, _E), lambda i: (0, 0, 0)),
            pl.BlockSpec((_L, _E), lambda i: (0, 0)),
            pl.BlockSpec((_E, D, D), lambda i: (0, 0, 0)),
            pl.BlockSpec((_E, D), lambda i: (0, 0)),
        ],
        out_specs=pl.BlockSpec((_BLK, D), lambda i: (i, 0)),
        out_shape=jax.ShapeDtypeStruct((N, D), jnp.float32),
        scratch_shapes=[pltpu.VMEM((_BLK, D), jnp.float32)],
        interpret=interpret,
    )(x, WgT, bg, WeT, be)
    return out.reshape(B, S, D)


# speedup vs baseline: 4.9567x; 4.9567x over previous
import jax
import jax.numpy as jnp
from jax.experimental import pallas as pl
from jax.experimental.pallas import tpu as pltpu

_L = 4
_E = 8
_D = 80
_BLK = 2048


def _is_int_0_8(v):
    vi = v.astype(jnp.int32)
    return (vi.astype(jnp.float32) == v) & (v >= 0.0) & (v <= 7.0)


def _screen_body(x_ref, wgt0_ref, bg_ref, flag_ref):
    pid = pl.program_id(0)

    @pl.when(pid == 0)
    def _init():
        bghit = jnp.any(_is_int_0_8(bg_ref[...]))
        flag_ref[...] = jnp.where(bghit, 1.0, 0.0) * jnp.ones((8, 128), jnp.float32)

    g = (jnp.dot(x_ref[...], wgt0_ref[...], preferred_element_type=jnp.float32)
         + bg_ref[0][None, :])
    hit = jnp.any(_is_int_0_8(g))

    @pl.when(hit)
    def _acc():
        flag_ref[...] = jnp.ones((8, 128), jnp.float32)


def _top2_coeffs(g):
    T = g.shape[0]
    ids = jax.lax.broadcasted_iota(jnp.int32, (T, _E), 1)
    idf = ids.astype(jnp.float32)

    v0 = jnp.max(g, axis=1, keepdims=True)
    idx0 = jnp.min(jnp.where(g == v0, ids, _E), axis=1, keepdims=True)
    g1 = jnp.where(ids == idx0, -jnp.inf, g)
    v1 = jnp.max(g1, axis=1, keepdims=True)
    idx1 = jnp.min(jnp.where(g1 == v1, ids, _E), axis=1, keepdims=True)

    idx0f = idx0.astype(jnp.float32)
    idx1f = idx1.astype(jnp.float32)
    return (jnp.where(v0 == idf, idx0f, 0.0)
            + jnp.where(v1 == idf, idx1f, 0.0))


def _moe_body(x_ref, wgt_ref, bg_ref, wet_ref, be_ref, o_ref, xs_ref):
    T = x_ref.shape[0]
    xs_ref[...] = x_ref[...]

    for layer in range(_L):
        x = xs_ref[...]
        g = (jnp.dot(x, wgt_ref[layer], preferred_element_type=jnp.float32)
             + bg_ref[layer][None, :])
        c = _top2_coeffs(g)
        dst = o_ref if layer == _L - 1 else xs_ref
        acc = jnp.zeros((T, _D), jnp.float32)
        for e in range(_E):
            eo = (jnp.dot(x, wet_ref[e], preferred_element_type=jnp.float32)
                  + be_ref[e][None, :])
            acc = acc + c[:, e:e + 1] * eo
        dst[...] = acc


def kernel(input_features, Wg, bg, We, be, interpret=False):
    B, S, D = input_features.shape
    N = B * S
    x = input_features.reshape(N, D)
    WgT = jnp.transpose(Wg, (0, 2, 1))
    WeT = jnp.transpose(We, (0, 2, 1))

    grid = (N // _BLK,)

    flags = pl.pallas_call(
        _screen_body,
        grid=grid,
        in_specs=[
            pl.BlockSpec((_BLK, D), lambda i: (i, 0)),
            pl.BlockSpec((D, _E), lambda i: (0, 0)),
            pl.BlockSpec((_L, _E), lambda i: (0, 0)),
        ],
        out_specs=pl.BlockSpec((8, 128), lambda i: (0, 0)),
        out_shape=jax.ShapeDtypeStruct((8, 128), jnp.float32),
        interpret=interpret,
    )(x, WgT[0], bg)

    def _full(x):
        return pl.pallas_call(
            _moe_body,
            grid=grid,
            in_specs=[
                pl.BlockSpec((_BLK, D), lambda i: (i, 0)),
                pl.BlockSpec((_L, D, _E), lambda i: (0, 0, 0)),
                pl.BlockSpec((_L, _E), lambda i: (0, 0)),
                pl.BlockSpec((_E, D, D), lambda i: (0, 0, 0)),
                pl.BlockSpec((_E, D), lambda i: (0, 0)),
            ],
            out_specs=pl.BlockSpec((_BLK, D), lambda i: (i, 0)),
            out_shape=jax.ShapeDtypeStruct((N, D), jnp.float32),
            scratch_shapes=[pltpu.VMEM((_BLK, D), jnp.float32)],
            interpret=interpret,
        )(x, WgT, bg, WeT, be)

    out = jax.lax.cond(jnp.any(flags != 0.0), _full,
                       lambda x: jnp.zeros((N, D), jnp.float32), x)
    return out.reshape(B, S, D)


# speedup vs baseline: 4.9604x; 1.0007x over previous
import jax
import jax.numpy as jnp
from jax.experimental import pallas as pl
from jax.experimental.pallas import tpu as pltpu

_L = 4
_E = 8
_D = 80
_BLK = 2048


def _is_int_0_8(v):
    vi = v.astype(jnp.int32)
    return (vi.astype(jnp.float32) == v) & (v >= 0.0) & (v <= 7.0)


def _screen_body(x_ref, wg0_ref, bg_ref, flag_ref):
    pid = pl.program_id(0)

    @pl.when(pid == 0)
    def _init():
        bghit = jnp.any(_is_int_0_8(bg_ref[...]))
        flag_ref[...] = jnp.where(bghit, 1.0, 0.0) * jnp.ones((8, 128), jnp.float32)

    g = jax.lax.dot_general(
        x_ref[...], wg0_ref[...],
        dimension_numbers=(((1,), (1,)), ((), ())),
        preferred_element_type=jnp.float32,
    ) + bg_ref[0][None, :]
    hit = jnp.any(_is_int_0_8(g))

    @pl.when(hit)
    def _acc():
        flag_ref[...] = jnp.ones((8, 128), jnp.float32)


def _top2_coeffs(g):
    T = g.shape[0]
    ids = jax.lax.broadcasted_iota(jnp.int32, (T, _E), 1)
    idf = ids.astype(jnp.float32)

    v0 = jnp.max(g, axis=1, keepdims=True)
    idx0 = jnp.min(jnp.where(g == v0, ids, _E), axis=1, keepdims=True)
    g1 = jnp.where(ids == idx0, -jnp.inf, g)
    v1 = jnp.max(g1, axis=1, keepdims=True)
    idx1 = jnp.min(jnp.where(g1 == v1, ids, _E), axis=1, keepdims=True)

    idx0f = idx0.astype(jnp.float32)
    idx1f = idx1.astype(jnp.float32)
    return (jnp.where(v0 == idf, idx0f, 0.0)
            + jnp.where(v1 == idf, idx1f, 0.0))


def _moe_body(x_ref, wgt_ref, bg_ref, wet_ref, be_ref, o_ref, xs_ref):
    T = x_ref.shape[0]
    xs_ref[...] = x_ref[...]

    for layer in range(_L):
        x = xs_ref[...]
        g = (jnp.dot(x, wgt_ref[layer], preferred_element_type=jnp.float32)
             + bg_ref[layer][None, :])
        c = _top2_coeffs(g)
        dst = o_ref if layer == _L - 1 else xs_ref
        acc = jnp.zeros((T, _D), jnp.float32)
        for e in range(_E):
            eo = (jnp.dot(x, wet_ref[e], preferred_element_type=jnp.float32)
                  + be_ref[e][None, :])
            acc = acc + c[:, e:e + 1] * eo
        dst[...] = acc


def kernel(input_features, Wg, bg, We, be, interpret=False):
    B, S, D = input_features.shape
    N = B * S
    x = input_features.reshape(N, D)

    grid = (N // _BLK,)

    flags = pl.pallas_call(
        _screen_body,
        grid=grid,
        in_specs=[
            pl.BlockSpec((_BLK, D), lambda i: (i, 0)),
            pl.BlockSpec((_E, D), lambda i: (0, 0)),
            pl.BlockSpec((_L, _E), lambda i: (0, 0)),
        ],
        out_specs=pl.BlockSpec((8, 128), lambda i: (0, 0)),
        out_shape=jax.ShapeDtypeStruct((8, 128), jnp.float32),
        interpret=interpret,
    )(x, Wg[0], bg)

    def _full(x):
        WgT = jnp.transpose(Wg, (0, 2, 1))
        WeT = jnp.transpose(We, (0, 2, 1))
        return pl.pallas_call(
            _moe_body,
            grid=grid,
            in_specs=[
                pl.BlockSpec((_BLK, D), lambda i: (i, 0)),
                pl.BlockSpec((_L, D, _E), lambda i: (0, 0, 0)),
                pl.BlockSpec((_L, _E), lambda i: (0, 0)),
                pl.BlockSpec((_E, D, D), lambda i: (0, 0, 0)),
                pl.BlockSpec((_E, D), lambda i: (0, 0)),
            ],
            out_specs=pl.BlockSpec((_BLK, D), lambda i: (i, 0)),
            out_shape=jax.ShapeDtypeStruct((N, D), jnp.float32),
            scratch_shapes=[pltpu.VMEM((_BLK, D), jnp.float32)],
            interpret=interpret,
        )(x, WgT, bg, WeT, be)

    out = jax.lax.cond(jnp.any(flags != 0.0), _full,
                       lambda x: jnp.zeros((N, D), jnp.float32), x)
    return out.reshape(B, S, D)
